# baseline (speedup 1.0000x reference)
import dataclasses
import functools

import jax
import jax.numpy as jnp
import numpy as np
from jax import lax
from jax.experimental import pallas as pl
from jax.experimental.pallas import tpu as pltpu
from jax.experimental.pallas import tpu_sc as plsc

F32 = jnp.float32
NC = 2
NS = 16
NW = NC * NS
SUB = 80
BN = 2000

_DN = lax.GatherDimensionNumbers(
    offset_dims=(), collapsed_slice_dims=(0,), start_index_map=(0,))


def _take16(v, idx):
    return lax.gather(v, idx.reshape(16, 1), _DN, (1,),
                      indices_are_sorted=False, unique_indices=False,
                      mode=lax.GatherScatterMode.PROMISE_IN_BOUNDS)


def _stage_a(x, pos, Wmp, bmp, Wsx, Wsp, Wdx, Wdp, p96, p32):
    N = x.shape[0]
    NB = N // BN
    dot = functools.partial(jnp.dot, preferred_element_type=F32,
                            precision=lax.Precision.DEFAULT)

    def body(x_ref, pos_ref, wmp_ref, bmp_ref, wsx_ref, wsp_ref, wdx_ref,
             wdp_ref, p96_ref, p32_ref, ts_ref, td_ref):
        p = dot(pos_ref[...], wmp_ref[...]) + bmp_ref[...]
        xb = x_ref[...]
        hs = dot(xb, wsx_ref[...]) + dot(p, wsp_ref[...])
        ad = dot(xb, wdx_ref[...]) + dot(p, wdp_ref[...])
        hp = hs[:, 0:64]
        as1 = hs[:, 64:72]
        ts = jnp.concatenate(
            [hp, jnp.exp(as1), jnp.exp(0.2 * as1)], axis=1)
        td = jnp.concatenate(
            [jnp.exp(ad), jnp.exp(0.2 * ad)], axis=1)
        ts_ref[...] = dot(ts, p96_ref[...]).astype(jnp.bfloat16)
        td_ref[...] = dot(td, p32_ref[...]).astype(jnp.bfloat16)

    return pl.pallas_call(
        body,
        grid=(NB,),
        in_specs=[
            pl.BlockSpec((BN, 128), lambda i: (i, 0)),
            pl.BlockSpec((BN, 16), lambda i: (i, 0)),
            pl.BlockSpec((16, 16), lambda i: (0, 0)),
            pl.BlockSpec((1, 16), lambda i: (0, 0)),
            pl.BlockSpec((128, 72), lambda i: (0, 0)),
            pl.BlockSpec((16, 72), lambda i: (0, 0)),
            pl.BlockSpec((128, 8), lambda i: (0, 0)),
            pl.BlockSpec((16, 8), lambda i: (0, 0)),
            pl.BlockSpec((80, 96), lambda i: (0, 0)),
            pl.BlockSpec((16, 32), lambda i: (0, 0)),
        ],
        out_specs=[
            pl.BlockSpec((BN, 96), lambda i: (i, 0)),
            pl.BlockSpec((BN, 32), lambda i: (i, 0)),
        ],
        out_shape=[
            jax.ShapeDtypeStruct((N, 96), jnp.bfloat16),
            jax.ShapeDtypeStruct((N, 32), jnp.bfloat16),
        ],
    )(x, pos, Wmp, bmp, Wsx, Wsp, Wdx, Wdp, p96, p32)


def _stage_b(acc1, b1, exp8, W2big, W2d, onehot40, p64, p32):
    NB = acc1.shape[1] // BN
    N = NB * BN
    dot = functools.partial(jnp.dot, preferred_element_type=F32,
                            precision=lax.Precision.DEFAULT)

    def body(acc_ref, b1_ref, exp8_ref, w2b_ref, w2d_ref, oh_ref,
             p64_ref, p32_ref, ts_ref, td_ref):
        s = acc_ref[0, :, 0:72] + acc_ref[1, :, 0:72]
        num = s[:, 0:64]
        den = s[:, 64:72]
        o1 = num / (dot(den, exp8_ref[...]) + 1e-16) + b1_ref[...]
        o1 = jnp.maximum(o1, 0.0) + (jnp.exp(jnp.minimum(o1, 0.0)) - 1.0)
        ts = dot(o1, w2b_ref[...]) + oh_ref[...]
        td = dot(o1, w2d_ref[...])
        ts_ref[...] = dot(ts, p64_ref[...]).astype(jnp.bfloat16)
        td_ref[...] = dot(td, p32_ref[...]).astype(jnp.bfloat16)

    return pl.pallas_call(
        body,
        grid=(NB,),
        in_specs=[
            pl.BlockSpec((2, BN, 128), lambda i: (0, i, 0)),
            pl.BlockSpec((1, 64), lambda i: (0, 0)),
            pl.BlockSpec((8, 64), lambda i: (0, 0)),
            pl.BlockSpec((64, 48), lambda i: (0, 0)),
            pl.BlockSpec((64, 16), lambda i: (0, 0)),
            pl.BlockSpec((1, 48), lambda i: (0, 0)),
            pl.BlockSpec((48, 64), lambda i: (0, 0)),
            pl.BlockSpec((16, 32), lambda i: (0, 0)),
        ],
        out_specs=[
            pl.BlockSpec((BN, 64), lambda i: (i, 0)),
            pl.BlockSpec((BN, 32), lambda i: (i, 0)),
        ],
        out_shape=[
            jax.ShapeDtypeStruct((N, 64), jnp.bfloat16),
            jax.ShapeDtypeStruct((N, 32), jnp.bfloat16),
        ],
    )(acc1, b1, exp8, W2big, W2d, onehot40, p64, p32)


def _stage_c(acc2, b2):
    NB = acc2.shape[1] // BN
    N = NB * BN

    def body(acc_ref, b2_ref, out_ref):
        s = acc_ref[0, :, 0:41] + acc_ref[1, :, 0:41]
        num = s[:, 0:40]
        den = s[:, 40:41]
        logits = num / (den + 1e-16) + b2_ref[...]
        m = jnp.max(logits, axis=1, keepdims=True)
        z = logits - m
        out_ref[...] = z - jnp.log(jnp.sum(jnp.exp(z), axis=1, keepdims=True))

    return pl.pallas_call(
        body,
        grid=(NB,),
        in_specs=[
            pl.BlockSpec((2, BN, 128), lambda i: (0, i, 0)),
            pl.BlockSpec((1, 40), lambda i: (0, 0)),
        ],
        out_specs=pl.BlockSpec((BN, 40), lambda i: (i, 0)),
        out_shape=jax.ShapeDtypeStruct((N, 40), F32),
    )(acc2, b2)


def _sc_compiler_params():
    cp = pltpu.CompilerParams(use_tc_tiling_on_sc=False)
    if "needs_layout_passes" in pltpu.CompilerParams.__dataclass_fields__:
        cp = dataclasses.replace(cp, needs_layout_passes=False)
    return cp


def _edge_pass(tsrc, tdst, ei3, zeros, layer):
    N, DG = tsrc.shape
    D = zeros.shape[1]
    E = ei3.shape[1] * SUB
    npad = zeros.shape[0]
    ept = E // NW
    nch = ept // SUB
    rpt = npad // NS
    mesh = plsc.VectorSubcoreMesh(core_axis_name="c", subcore_axis_name="s")

    @functools.partial(
        pl.kernel,
        mesh=mesh,
        compiler_params=_sc_compiler_params(),
        out_type=jax.ShapeDtypeStruct((NC, npad, 128), F32),
        scratch_types=[
            pltpu.VMEM((nch, SUB), jnp.int32),
            pltpu.VMEM((nch, SUB), jnp.int32),
            pltpu.VMEM((2, SUB, DG), jnp.bfloat16),
            pltpu.VMEM((2, SUB, 32), jnp.bfloat16),
            pltpu.VMEM((2, SUB, D), F32),
            pltpu.VMEM_SHARED((npad, D), F32),
            pltpu.SemaphoreType.DMA,
            pltpu.SemaphoreType.DMA,
            pltpu.SemaphoreType.DMA,
            pltpu.SemaphoreType.DMA,
            pltpu.SemaphoreType.DMA,
            pltpu.SemaphoreType.DMA,
            pltpu.SemaphoreType.DMA,
            pltpu.SemaphoreType.DMA,
            pltpu.SemaphoreType.DMA,
            pltpu.SemaphoreType.DMA,
        ],
    )
    def k(tsrc_hbm, tdst_hbm, ei_hbm, z_hbm, out_hbm,
          sidx_all, didx_all, srows_v, drows_v, msg_v, acc_sh,
          gs0, gs1, gd0, gd1, ss0, ss1, gh0, gh1, sh0, sh1):
        cid = lax.axis_index("c")
        sid = lax.axis_index("s")
        wid = cid * NS + sid
        gsem = (gs0, gs1)
        gdem = (gd0, gd1)
        ssem = (ss0, ss1)
        ghem = (gh0, gh1)
        shem = (sh0, sh1)
        H = SUB // 2

        pltpu.sync_copy(z_hbm.at[pl.ds(sid * rpt, rpt)],
                        acc_sh.at[pl.ds(sid * rpt, rpt)])
        pltpu.sync_copy(ei_hbm.at[0, pl.ds(wid * nch, nch)], sidx_all)
        pltpu.sync_copy(ei_hbm.at[1, pl.ds(wid * nch, nch)], didx_all)
        plsc.subcore_barrier()

        rot8 = jnp.bitwise_and(lax.iota(jnp.int32, 16) + 8, 15)
        nine = jnp.full((16,), 9, jnp.int32)
        zero16 = jnp.zeros((16,), jnp.int32)

        def start_gather(ck, b):
            pltpu.async_copy(tsrc_hbm.at[sidx_all.at[ck, pl.ds(0, H)]],
                             srows_v.at[b, pl.ds(0, H)], gsem[b])
            pltpu.async_copy(tsrc_hbm.at[sidx_all.at[ck, pl.ds(H, H)]],
                             srows_v.at[b, pl.ds(H, H)], ghem[b])
            pltpu.async_copy(tdst_hbm.at[didx_all.at[ck]], drows_v.at[b],
                             gdem[b])

        def wait_gather(ck, b):
            pltpu.make_async_copy(tsrc_hbm.at[sidx_all.at[ck, pl.ds(0, H)]],
                                  srows_v.at[b, pl.ds(0, H)], gsem[b]).wait()
            pltpu.make_async_copy(tsrc_hbm.at[sidx_all.at[ck, pl.ds(H, H)]],
                                  srows_v.at[b, pl.ds(H, H)], ghem[b]).wait()
            pltpu.make_async_copy(tdst_hbm.at[didx_all.at[ck]],
                                  drows_v.at[b], gdem[b]).wait()

        def start_scatter(ck, b):
            pltpu.async_copy(msg_v.at[b, pl.ds(0, H)],
                             acc_sh.at[didx_all.at[ck, pl.ds(0, H)]],
                             ssem[b], add=True)
            pltpu.async_copy(msg_v.at[b, pl.ds(H, H)],
                             acc_sh.at[didx_all.at[ck, pl.ds(H, H)]],
                             shem[b], add=True)

        def wait_scatter(ck, b):
            pltpu.make_async_copy(msg_v.at[b, pl.ds(0, H)],
                                  acc_sh.at[didx_all.at[ck, pl.ds(0, H)]],
                                  ssem[b]).wait()
            pltpu.make_async_copy(msg_v.at[b, pl.ds(H, H)],
                                  acc_sh.at[didx_all.at[ck, pl.ds(H, H)]],
                                  shem[b]).wait()

        def compute(b):
            sb = srows_v.at[b]
            db = drows_v.at[b]
            mb = msg_v.at[b]
            def up(chunk):
                return plsc.unpack(chunk, format=plsc.PackFormat.INTERLEAVED,
                                   preferred_element_type=F32)

            if layer == 1:
                @plsc.parallel_loop(0, SUB, unroll=4)
                def _edge(e):
                    g0, g1 = up(sb[e, pl.ds(0, 32)])
                    g2, g3 = up(sb[e, pl.ds(32, 32)])
                    asv, _unused = up(sb[e, pl.ds(64, 32)])
                    adv, _unused2 = up(db[e, pl.ds(0, 32)])
                    u = asv * adv
                    w = jnp.maximum(u, _take16(u, rot8))
                    mb[e, pl.ds(0, 16)] = g0 * w
                    mb[e, pl.ds(16, 16)] = g1 * w
                    mb[e, pl.ds(32, 16)] = g2 * w
                    mb[e, pl.ds(48, 16)] = g3 * w
                    mb[e, pl.ds(64, 16)] = w
            else:
                @plsc.parallel_loop(0, SUB, unroll=4)
                def _edge(e):
                    g0, g1 = up(sb[e, pl.ds(0, 32)])
                    g2, _unused = up(sb[e, pl.ds(32, 32)])
                    adv, _unused2 = up(db[e, pl.ds(0, 32)])
                    a = _take16(g2, nine) + _take16(adv, zero16)
                    w = jnp.exp(jnp.maximum(a, 0.2 * a))
                    mb[e, pl.ds(0, 16)] = g0 * w
                    mb[e, pl.ds(16, 16)] = g1 * w
                    mb[e, pl.ds(32, 16)] = g2 * w

        start_gather(0, 0)

        @pl.loop(0, (nch - 1) // 2)
        def _pair(kk):
            for b in range(2):
                ck = 2 * kk + b
                wait_gather(ck, b)
                start_gather(ck + 1, 1 - b)

                @pl.when(kk >= 1)
                def _():
                    wait_scatter(ck, b)

                compute(b)
                start_scatter(ck, b)

        wait_gather(nch - 1, 0)
        wait_scatter(nch - 1, 0)
        compute(0)
        start_scatter(nch - 1, 0)
        wait_scatter(nch - 1, 0)
        wait_scatter(nch - 2, 1)

        plsc.subcore_barrier()
        pltpu.sync_copy(acc_sh.at[pl.ds(sid * rpt, rpt)],
                        out_hbm.at[cid, pl.ds(sid * rpt, rpt), pl.ds(0, D)])

    return k(tsrc, tdst, ei3, zeros)


def kernel(x, pos_encoding, edge_index, W_mp, b_mp, W1, att_src1, att_dst1,
           b1, W2, att_src2, att_dst2, b2):
    N = x.shape[0]
    E = edge_index.shape[1]

    ei3 = edge_index.astype(jnp.int32).reshape(2, E // SUB, SUB)
    npad = ((N + NS * 8 - 1) // (NS * 8)) * NS * 8

    blockdiag = (jnp.arange(64)[:, None] // 8
                 == jnp.arange(8)[None, :]).astype(F32)
    A_s = blockdiag * att_src1.reshape(64, 1)
    A_d = blockdiag * att_dst1.reshape(64, 1)
    perm = (jnp.arange(64) % 8) * 8 + jnp.arange(64) // 8
    Wsrc = jnp.concatenate([W1[:, perm], W1 @ A_s], 1)
    Wdst = W1 @ A_d
    W2big = jnp.concatenate([W2, jnp.zeros((64, 1), F32),
                             W2 @ att_src2.reshape(40, 1),
                             jnp.zeros((64, 6), F32)], 1)
    W2d = jnp.concatenate([W2 @ att_dst2.reshape(40, 1),
                           jnp.zeros((64, 15), F32)], 1)
    W2big = W2big.reshape(8, 8, 48).transpose(1, 0, 2).reshape(64, 48)
    W2d = W2d.reshape(8, 8, 16).transpose(1, 0, 2).reshape(64, 16)
    b1i = b1.reshape(8, 8).T.reshape(64)
    onehot40 = (jnp.arange(48) == 40).astype(F32).reshape(1, 48)
    exp8 = (jnp.arange(64)[None, :] % 8
            == jnp.arange(8)[:, None]).astype(F32)

    def interleave_matrix(nl, npk, pairs):
        p2l = np.full(npk, -1)
        for base, a0, b0 in pairs:
            for i in range(16):
                p2l[base + 2 * i] = a0 + i
                if b0 is not None:
                    p2l[base + 2 * i + 1] = b0 + i
        return jnp.asarray(p2l[None, :] == np.arange(nl)[:, None], F32)

    p96 = interleave_matrix(80, 96, [(0, 0, 16), (32, 32, 48), (64, 64, None)])
    p64 = interleave_matrix(48, 64, [(0, 0, 16), (32, 32, None)])
    p32 = interleave_matrix(16, 32, [(0, 0, None)])

    tsrc1, tdst1 = _stage_a(x, pos_encoding, W_mp, b_mp.reshape(1, 16),
                            Wsrc[:128], Wsrc[128:], Wdst[:128], Wdst[128:],
                            p96, p32)
    acc1 = _edge_pass(tsrc1, tdst1, ei3,
                      jnp.zeros((npad, 80), F32), layer=1)
    tsrc2, tdst2 = _stage_b(acc1, b1i.reshape(1, 64), exp8, W2big, W2d,
                            onehot40, p64, p32)
    acc2 = _edge_pass(tsrc2, tdst2, ei3,
                      jnp.zeros((npad, 48), F32), layer=2)
    return _stage_c(acc2, b2.reshape(1, 40))


# speedup vs baseline: 1.0020x; 1.0020x over previous
import dataclasses
import functools

import jax
import jax.numpy as jnp
import numpy as np
from jax import lax
from jax.experimental import pallas as pl
from jax.experimental.pallas import tpu as pltpu
from jax.experimental.pallas import tpu_sc as plsc

F32 = jnp.float32
NC = 2
NS = 16
NW = NC * NS
SUB = 80
BN = 2000

_DN = lax.GatherDimensionNumbers(
    offset_dims=(), collapsed_slice_dims=(0,), start_index_map=(0,))


def _take16(v, idx):
    return lax.gather(v, idx.reshape(16, 1), _DN, (1,),
                      indices_are_sorted=False, unique_indices=False,
                      mode=lax.GatherScatterMode.PROMISE_IN_BOUNDS)


def _stage_a(x, pos, Wmp, bmp, Wsx, Wsp, Wdx, Wdp, p96, p32):
    N = x.shape[0]
    NB = N // BN
    dot = functools.partial(jnp.dot, preferred_element_type=F32,
                            precision=lax.Precision.DEFAULT)

    def body(x_ref, pos_ref, wmp_ref, bmp_ref, wsx_ref, wsp_ref, wdx_ref,
             wdp_ref, p96_ref, p32_ref, ts_ref, td_ref):
        p = dot(pos_ref[...], wmp_ref[...]) + bmp_ref[...]
        xb = x_ref[...]
        hs = dot(xb, wsx_ref[...]) + dot(p, wsp_ref[...])
        ad = dot(xb, wdx_ref[...]) + dot(p, wdp_ref[...])
        hp = hs[:, 0:64]
        as1 = hs[:, 64:72]
        ts = jnp.concatenate(
            [hp, jnp.exp(as1), jnp.exp(0.2 * as1)], axis=1)
        td = jnp.concatenate(
            [jnp.exp(ad), jnp.exp(0.2 * ad)], axis=1)
        ts_ref[...] = dot(ts, p96_ref[...]).astype(jnp.bfloat16)
        td_ref[...] = dot(td, p32_ref[...]).astype(jnp.bfloat16)

    return pl.pallas_call(
        body,
        grid=(NB,),
        in_specs=[
            pl.BlockSpec((BN, 128), lambda i: (i, 0)),
            pl.BlockSpec((BN, 16), lambda i: (i, 0)),
            pl.BlockSpec((16, 16), lambda i: (0, 0)),
            pl.BlockSpec((1, 16), lambda i: (0, 0)),
            pl.BlockSpec((128, 72), lambda i: (0, 0)),
            pl.BlockSpec((16, 72), lambda i: (0, 0)),
            pl.BlockSpec((128, 8), lambda i: (0, 0)),
            pl.BlockSpec((16, 8), lambda i: (0, 0)),
            pl.BlockSpec((80, 96), lambda i: (0, 0)),
            pl.BlockSpec((16, 32), lambda i: (0, 0)),
        ],
        out_specs=[
            pl.BlockSpec((BN, 96), lambda i: (i, 0)),
            pl.BlockSpec((BN, 32), lambda i: (i, 0)),
        ],
        out_shape=[
            jax.ShapeDtypeStruct((N, 96), jnp.bfloat16),
            jax.ShapeDtypeStruct((N, 32), jnp.bfloat16),
        ],
    )(x, pos, Wmp, bmp, Wsx, Wsp, Wdx, Wdp, p96, p32)


def _stage_b(acc1, b1, exp8, W2big, W2d, onehot40, p64, p32):
    NB = acc1.shape[1] // BN
    N = NB * BN
    dot = functools.partial(jnp.dot, preferred_element_type=F32,
                            precision=lax.Precision.DEFAULT)

    def body(acc_ref, b1_ref, exp8_ref, w2b_ref, w2d_ref, oh_ref,
             p64_ref, p32_ref, ts_ref, td_ref):
        s = acc_ref[0, :, 0:72] + acc_ref[1, :, 0:72]
        num = s[:, 0:64]
        den = s[:, 64:72]
        o1 = num / (dot(den, exp8_ref[...]) + 1e-16) + b1_ref[...]
        o1 = jnp.maximum(o1, 0.0) + (jnp.exp(jnp.minimum(o1, 0.0)) - 1.0)
        ts = dot(o1, w2b_ref[...]) + oh_ref[...]
        td = dot(o1, w2d_ref[...])
        ts_ref[...] = dot(ts, p64_ref[...]).astype(jnp.bfloat16)
        td_ref[...] = dot(td, p32_ref[...]).astype(jnp.bfloat16)

    return pl.pallas_call(
        body,
        grid=(NB,),
        in_specs=[
            pl.BlockSpec((2, BN, 128), lambda i: (0, i, 0)),
            pl.BlockSpec((1, 64), lambda i: (0, 0)),
            pl.BlockSpec((8, 64), lambda i: (0, 0)),
            pl.BlockSpec((64, 48), lambda i: (0, 0)),
            pl.BlockSpec((64, 16), lambda i: (0, 0)),
            pl.BlockSpec((1, 48), lambda i: (0, 0)),
            pl.BlockSpec((48, 64), lambda i: (0, 0)),
            pl.BlockSpec((16, 32), lambda i: (0, 0)),
        ],
        out_specs=[
            pl.BlockSpec((BN, 64), lambda i: (i, 0)),
            pl.BlockSpec((BN, 32), lambda i: (i, 0)),
        ],
        out_shape=[
            jax.ShapeDtypeStruct((N, 64), jnp.bfloat16),
            jax.ShapeDtypeStruct((N, 32), jnp.bfloat16),
        ],
    )(acc1, b1, exp8, W2big, W2d, onehot40, p64, p32)


def _stage_c(acc2, b2):
    NB = acc2.shape[1] // BN
    N = NB * BN

    def body(acc_ref, b2_ref, out_ref):
        s = acc_ref[0, :, 0:41] + acc_ref[1, :, 0:41]
        num = s[:, 0:40]
        den = s[:, 40:41]
        logits = num / (den + 1e-16) + b2_ref[...]
        m = jnp.max(logits, axis=1, keepdims=True)
        z = logits - m
        out_ref[...] = z - jnp.log(jnp.sum(jnp.exp(z), axis=1, keepdims=True))

    return pl.pallas_call(
        body,
        grid=(NB,),
        in_specs=[
            pl.BlockSpec((2, BN, 128), lambda i: (0, i, 0)),
            pl.BlockSpec((1, 40), lambda i: (0, 0)),
        ],
        out_specs=pl.BlockSpec((BN, 40), lambda i: (i, 0)),
        out_shape=jax.ShapeDtypeStruct((N, 40), F32),
    )(acc2, b2)


def _sc_compiler_params():
    cp = pltpu.CompilerParams(use_tc_tiling_on_sc=False)
    if "needs_layout_passes" in pltpu.CompilerParams.__dataclass_fields__:
        cp = dataclasses.replace(cp, needs_layout_passes=False)
    return cp


def _edge_pass(tsrc, tdst, ei3, zeros, layer):
    N, DG = tsrc.shape
    D = zeros.shape[1]
    E = ei3.shape[1] * SUB
    npad = zeros.shape[0]
    ept = E // NW
    nch = ept // SUB
    rpt = npad // NS
    mesh = plsc.VectorSubcoreMesh(core_axis_name="c", subcore_axis_name="s")

    @functools.partial(
        pl.kernel,
        mesh=mesh,
        compiler_params=_sc_compiler_params(),
        out_type=jax.ShapeDtypeStruct((NC, npad, 128), F32),
        scratch_types=[
            pltpu.VMEM((nch, SUB), jnp.int32),
            pltpu.VMEM((nch, SUB), jnp.int32),
            pltpu.VMEM((2, SUB, DG), jnp.bfloat16),
            pltpu.VMEM((2, SUB, 32), jnp.bfloat16),
            pltpu.VMEM((2, SUB, D), F32),
            pltpu.VMEM_SHARED((npad, D), F32),
            pltpu.SemaphoreType.DMA,
            pltpu.SemaphoreType.DMA,
            pltpu.SemaphoreType.DMA,
            pltpu.SemaphoreType.DMA,
            pltpu.SemaphoreType.DMA,
            pltpu.SemaphoreType.DMA,
            pltpu.SemaphoreType.DMA,
            pltpu.SemaphoreType.DMA,
        ],
    )
    def k(tsrc_hbm, tdst_hbm, ei_hbm, z_hbm, out_hbm,
          sidx_all, didx_all, srows_v, drows_v, msg_v, acc_sh,
          gs0, gs1, gd0, gd1, ss0, ss1, gh0, gh1):
        cid = lax.axis_index("c")
        sid = lax.axis_index("s")
        wid = cid * NS + sid
        gsem = (gs0, gs1)
        gdem = (gd0, gd1)
        ssem = (ss0, ss1)
        ghem = (gh0, gh1)
        H = SUB // 2

        pltpu.sync_copy(z_hbm.at[pl.ds(sid * rpt, rpt)],
                        acc_sh.at[pl.ds(sid * rpt, rpt)])
        pltpu.sync_copy(ei_hbm.at[0, pl.ds(wid * nch, nch)], sidx_all)
        pltpu.sync_copy(ei_hbm.at[1, pl.ds(wid * nch, nch)], didx_all)
        plsc.subcore_barrier()

        rot8 = jnp.bitwise_and(lax.iota(jnp.int32, 16) + 8, 15)
        nine = jnp.full((16,), 9, jnp.int32)
        zero16 = jnp.zeros((16,), jnp.int32)

        def start_gather(ck, b):
            pltpu.async_copy(tsrc_hbm.at[sidx_all.at[ck, pl.ds(0, H)]],
                             srows_v.at[b, pl.ds(0, H)], gsem[b])
            pltpu.async_copy(tsrc_hbm.at[sidx_all.at[ck, pl.ds(H, H)]],
                             srows_v.at[b, pl.ds(H, H)], ghem[b])
            pltpu.async_copy(tdst_hbm.at[didx_all.at[ck]], drows_v.at[b],
                             gdem[b])

        def wait_gather(ck, b):
            pltpu.make_async_copy(tsrc_hbm.at[sidx_all.at[ck, pl.ds(0, H)]],
                                  srows_v.at[b, pl.ds(0, H)], gsem[b]).wait()
            pltpu.make_async_copy(tsrc_hbm.at[sidx_all.at[ck, pl.ds(H, H)]],
                                  srows_v.at[b, pl.ds(H, H)], ghem[b]).wait()
            pltpu.make_async_copy(tdst_hbm.at[didx_all.at[ck]],
                                  drows_v.at[b], gdem[b]).wait()

        def start_scatter(ck, b):
            pltpu.async_copy(msg_v.at[b], acc_sh.at[didx_all.at[ck]],
                             ssem[b], add=True)

        def wait_scatter(ck, b):
            pltpu.make_async_copy(msg_v.at[b], acc_sh.at[didx_all.at[ck]],
                                  ssem[b]).wait()

        def compute(b):
            sb = srows_v.at[b]
            db = drows_v.at[b]
            mb = msg_v.at[b]
            def up(chunk):
                return plsc.unpack(chunk, format=plsc.PackFormat.INTERLEAVED,
                                   preferred_element_type=F32)

            if layer == 1:
                @plsc.parallel_loop(0, SUB, unroll=4)
                def _edge(e):
                    g0, g1 = up(sb[e, pl.ds(0, 32)])
                    g2, g3 = up(sb[e, pl.ds(32, 32)])
                    asv, _unused = up(sb[e, pl.ds(64, 32)])
                    adv, _unused2 = up(db[e, pl.ds(0, 32)])
                    u = asv * adv
                    w = jnp.maximum(u, _take16(u, rot8))
                    mb[e, pl.ds(0, 16)] = g0 * w
                    mb[e, pl.ds(16, 16)] = g1 * w
                    mb[e, pl.ds(32, 16)] = g2 * w
                    mb[e, pl.ds(48, 16)] = g3 * w
                    mb[e, pl.ds(64, 16)] = w
            else:
                @plsc.parallel_loop(0, SUB, unroll=4)
                def _edge(e):
                    g0, g1 = up(sb[e, pl.ds(0, 32)])
                    g2, _unused = up(sb[e, pl.ds(32, 32)])
                    adv, _unused2 = up(db[e, pl.ds(0, 32)])
                    a = _take16(g2, nine) + _take16(adv, zero16)
                    w = jnp.exp(jnp.maximum(a, 0.2 * a))
                    mb[e, pl.ds(0, 16)] = g0 * w
                    mb[e, pl.ds(16, 16)] = g1 * w
                    mb[e, pl.ds(32, 16)] = g2 * w

        start_gather(0, 0)

        @pl.loop(0, (nch - 1) // 2)
        def _pair(kk):
            for b in range(2):
                ck = 2 * kk + b
                wait_gather(ck, b)
                start_gather(ck + 1, 1 - b)

                @pl.when(kk >= 1)
                def _():
                    wait_scatter(ck, b)

                compute(b)
                start_scatter(ck, b)

        wait_gather(nch - 1, 0)
        wait_scatter(nch - 1, 0)
        compute(0)
        start_scatter(nch - 1, 0)
        wait_scatter(nch - 1, 0)
        wait_scatter(nch - 2, 1)

        plsc.subcore_barrier()
        pltpu.sync_copy(acc_sh.at[pl.ds(sid * rpt, rpt)],
                        out_hbm.at[cid, pl.ds(sid * rpt, rpt), pl.ds(0, D)])

    return k(tsrc, tdst, ei3, zeros)


def kernel(x, pos_encoding, edge_index, W_mp, b_mp, W1, att_src1, att_dst1,
           b1, W2, att_src2, att_dst2, b2):
    N = x.shape[0]
    E = edge_index.shape[1]

    ei3 = edge_index.astype(jnp.int32).reshape(2, E // SUB, SUB)
    npad = ((N + NS * 8 - 1) // (NS * 8)) * NS * 8

    blockdiag = (jnp.arange(64)[:, None] // 8
                 == jnp.arange(8)[None, :]).astype(F32)
    A_s = blockdiag * att_src1.reshape(64, 1)
    A_d = blockdiag * att_dst1.reshape(64, 1)
    perm = (jnp.arange(64) % 8) * 8 + jnp.arange(64) // 8
    Wsrc = jnp.concatenate([W1[:, perm], W1 @ A_s], 1)
    Wdst = W1 @ A_d
    W2big = jnp.concatenate([W2, jnp.zeros((64, 1), F32),
                             W2 @ att_src2.reshape(40, 1),
                             jnp.zeros((64, 6), F32)], 1)
    W2d = jnp.concatenate([W2 @ att_dst2.reshape(40, 1),
                           jnp.zeros((64, 15), F32)], 1)
    W2big = W2big.reshape(8, 8, 48).transpose(1, 0, 2).reshape(64, 48)
    W2d = W2d.reshape(8, 8, 16).transpose(1, 0, 2).reshape(64, 16)
    b1i = b1.reshape(8, 8).T.reshape(64)
    onehot40 = (jnp.arange(48) == 40).astype(F32).reshape(1, 48)
    exp8 = (jnp.arange(64)[None, :] % 8
            == jnp.arange(8)[:, None]).astype(F32)

    def interleave_matrix(nl, npk, pairs):
        p2l = np.full(npk, -1)
        for base, a0, b0 in pairs:
            for i in range(16):
                p2l[base + 2 * i] = a0 + i
                if b0 is not None:
                    p2l[base + 2 * i + 1] = b0 + i
        return jnp.asarray(p2l[None, :] == np.arange(nl)[:, None], F32)

    p96 = interleave_matrix(80, 96, [(0, 0, 16), (32, 32, 48), (64, 64, None)])
    p64 = interleave_matrix(48, 64, [(0, 0, 16), (32, 32, None)])
    p32 = interleave_matrix(16, 32, [(0, 0, None)])

    tsrc1, tdst1 = _stage_a(x, pos_encoding, W_mp, b_mp.reshape(1, 16),
                            Wsrc[:128], Wsrc[128:], Wdst[:128], Wdst[128:],
                            p96, p32)
    acc1 = _edge_pass(tsrc1, tdst1, ei3,
                      jnp.zeros((npad, 80), F32), layer=1)
    tsrc2, tdst2 = _stage_b(acc1, b1i.reshape(1, 64), exp8, W2big, W2d,
                            onehot40, p64, p32)
    acc2 = _edge_pass(tsrc2, tdst2, ei3,
                      jnp.zeros((npad, 48), F32), layer=2)
    return _stage_c(acc2, b2.reshape(1, 40))
